# baseline (speedup 1.0000x reference)
import jax
import jax.numpy as jnp
from jax import lax
from jax.experimental import pallas as pl
from jax.experimental.pallas import tpu as pltpu
from jax.experimental.pallas import tpu_sc as plsc

_D = 100
_DP = 128
_LANE = 16
_HALF = 100


def _sc_gather_max(x2, table, *, num_workers, rows_per_worker):
    nc = 2

    def body(x_hbm, tbl_hbm, out_hbm, xv, buf0, buf1, buf2, buf3,
             acc_v, sem0, sem1, sem2, sem3):
        wid = lax.axis_index("s") * nc + lax.axis_index("c")
        hbase = wid * (2 * rows_per_worker)
        pltpu.sync_copy(x_hbm.at[pl.ds(hbase, 2 * rows_per_worker)], xv)

        bufs = (buf0, buf1, buf2, buf3)
        sems = (sem0, sem1, sem2, sem3)

        def gather(half_idx, k):
            pltpu.make_async_copy(tbl_hbm.at[xv.at[half_idx]],
                                  bufs[k], sems[k]).start()

        def gwait(k):
            pltpu.make_async_copy(tbl_hbm.at[xv.at[0]],
                                  bufs[k], sems[k]).wait()

        def accum(buf, sums):
            def one(l, ss):
                return tuple(ss[j] + buf[l, pl.ds(j * _LANE, _LANE)]
                             for j in range(7))
            return lax.fori_loop(0, _HALF, one, sums, unroll=4)

        for k in range(4):
            gather(k, k)

        zeros = tuple(jnp.zeros((_LANE,), jnp.float32) for _ in range(7))
        neg = tuple(jnp.full((_LANE,), -jnp.inf, jnp.float32) for _ in range(7))

        def pair_body(p, maxes):
            more = p + 1 < rows_per_worker // 2

            def step(k, sums):
                gwait(k)
                sums = accum(bufs[k], sums)

                @pl.when(more)
                def _():
                    gather(4 * p + 4 + k, k)
                return sums

            s = step(0, zeros)
            s = step(1, s)
            maxes = tuple(jnp.maximum(a, b) for a, b in zip(maxes, s))
            s = step(2, zeros)
            s = step(3, s)
            return tuple(jnp.maximum(a, b) for a, b in zip(maxes, s))

        maxes = lax.fori_loop(0, rows_per_worker // 2, pair_body, neg)

        acc_v[pl.ds(112, _LANE)] = jnp.zeros((_LANE,), jnp.float32)
        for j in range(7):
            acc_v[pl.ds(j * _LANE, _LANE)] = maxes[j]
        pltpu.sync_copy(acc_v, out_hbm.at[wid])

    mesh = plsc.VectorSubcoreMesh(core_axis_name="c", subcore_axis_name="s")
    return pl.kernel(
        body,
        mesh=mesh,
        compiler_params=pltpu.CompilerParams(use_tc_tiling_on_sc=False),
        out_type=jax.ShapeDtypeStruct((num_workers, _DP), jnp.float32),
        scratch_types=[
            pltpu.VMEM((2 * rows_per_worker, _HALF), jnp.int32),
            pltpu.VMEM((_HALF, _DP), jnp.float32),
            pltpu.VMEM((_HALF, _DP), jnp.float32),
            pltpu.VMEM((_HALF, _DP), jnp.float32),
            pltpu.VMEM((_HALF, _DP), jnp.float32),
            pltpu.VMEM((_DP,), jnp.float32),
            pltpu.SemaphoreType.DMA,
            pltpu.SemaphoreType.DMA,
            pltpu.SemaphoreType.DMA,
            pltpu.SemaphoreType.DMA,
        ],
    )(x2, table)


def _tc_pad_table(table):
    v = table.shape[0]
    blk = 24576
    grid = (v + blk - 1) // blk

    def body(t_ref, o_ref):
        o_ref[...] = jnp.pad(t_ref[...], ((0, 0), (0, _DP - _D)))

    return pl.pallas_call(
        body,
        grid=(grid,),
        in_specs=[pl.BlockSpec((blk, _D), lambda i: (i, 0))],
        out_specs=pl.BlockSpec((blk, _DP), lambda i: (i, 0)),
        out_shape=jax.ShapeDtypeStruct((v, _DP), jnp.float32),
        compiler_params=pltpu.CompilerParams(
            vmem_limit_bytes=56 * 1024 * 1024),
    )(table)


def _tc_head(smax, w_t, b2, y2):
    def body(s_ref, w_ref, b_ref, y_ref, pred_ref, loss_ref):
        s = s_ref[...]
        m = jnp.max(s, axis=0, keepdims=True)
        col = lax.broadcasted_iota(jnp.int32, m.shape, 1)
        m = jnp.where(col < _D, m, 0.0)
        h = jnp.where(m > 0, m, 0.01 * m)
        logits = jnp.dot(h, w_ref[...],
                         preferred_element_type=jnp.float32) + b_ref[...]
        e = jnp.exp(logits - jnp.max(logits, axis=1, keepdims=True))
        pred = e / jnp.sum(e, axis=1, keepdims=True)
        pred_ref[...] = pred
        pe = jnp.exp(pred - jnp.max(pred, axis=1, keepdims=True))
        lse = jnp.log(jnp.sum(pe, axis=1, keepdims=True)) + jnp.max(
            pred, axis=1, keepdims=True)
        picked = jnp.sum(pred * y_ref[...], axis=1, keepdims=True)
        loss_ref[...] = lse - picked

    return pl.pallas_call(
        body,
        out_shape=(
            jax.ShapeDtypeStruct((1, 4), jnp.float32),
            jax.ShapeDtypeStruct((1, 1), jnp.float32),
        ),
    )(smax, w_t, b2, y2)


@jax.jit
def kernel(x, y, E_td, w_out, b_out):
    bsz, seq = x.shape
    num_workers = 32
    rows_per_worker = bsz // num_workers
    x2 = x.reshape(bsz * 2, seq // 2)
    tblp = _tc_pad_table(E_td)
    smax = _sc_gather_max(x2, tblp, num_workers=num_workers,
                          rows_per_worker=rows_per_worker)
    w_t = jnp.pad(w_out, ((0, 0), (0, _DP - _D))).T
    pred2, loss2 = _tc_head(smax, w_t, b_out.reshape(1, 4), y.reshape(1, 4))
    return pred2.reshape(4), loss2.reshape(())
